# baseline (speedup 1.0000x reference)
import jax
import jax.numpy as jnp
from jax import lax
from jax.experimental import pallas as pl
from jax.experimental.pallas import tpu as pltpu
from jax.experimental.pallas import tpu_sc as plsc

N_NODES = 100000
N_EDGES = 3200000
CUTOFF = 10.0
ON_CUT = 0.25 * CUTOFF
OFF_CUT = 0.75 * CUTOFF
ALPHA = 4.0 / CUTOFF + 0.001

NW = 32
NPT = N_NODES // NW
ACC_PAD = 3136
CHUNK = 4000
LANES = 16


def _pair_term(r, qi, qj):
    one = jnp.float32(1.0)
    x = (r - jnp.float32(ON_CUT)) * jnp.float32(1.0 / (OFF_CUT - ON_CUT))
    d = (x + x - one) / (x - x * x)
    d = jnp.minimum(d, jnp.float32(80.0))
    f = one / (one + jnp.exp(d))
    f = jnp.where(x <= 0.0, one, jnp.where(x >= 1.0, jnp.float32(0.0), f))
    rr = r * r
    u = rr + one
    ui = plsc.bitcast(u, jnp.int32)
    yi = jnp.int32(0x5F3759DF) - (ui >> 1)
    y = plsc.bitcast(yi, jnp.float32)
    y = y * (jnp.float32(1.5) - jnp.float32(0.5) * u * y * y)
    z = jnp.float32(ALPHA) * r
    ct = one / (r + jnp.float32(0.47047 * ALPHA) * rr)
    t = r * ct
    coul = (one + jnp.float32(0.47047) * z) * ct
    p = (jnp.float32(0.3480242)
         + t * (jnp.float32(-0.0958798) + t * jnp.float32(0.7478556)))
    erfc = t * p * jnp.exp(-z * z)
    return qi * qj * (coul + f * (y - coul)) * erfc


def _body(qa_hbm, r_hbm, ii_hbm, jj_hbm, bnd_hbm, out_hbm,
          qa_v, acc_v, bnd_v,
          r_a, ii_a, jj_a, r_b, ii_b, jj_b, sem_a, sem_b):
    cid = lax.axis_index("c")
    sid = lax.axis_index("s")
    wid = cid * 16 + sid

    pltpu.sync_copy(qa_hbm, qa_v)
    pltpu.sync_copy(bnd_hbm, bnd_v)

    def zero_body(i, carry):
        acc_v[pl.ds(i * LANES, LANES)] = jnp.zeros((LANES,), jnp.float32)
        return carry
    lax.fori_loop(0, ACC_PAD // LANES, zero_body, 0)

    lanes = lax.iota(jnp.int32, 16)

    def extract(pos):
        acc = jnp.zeros((LANES,), jnp.int32)
        for k in range(3):
            bk = bnd_v[pl.ds(k * LANES, LANES)]
            acc = acc + jnp.where(lanes + jnp.int32(k * LANES) == pos, bk,
                                  jnp.int32(0))
        return jnp.max(acc)

    e_start = extract(wid)
    e_end = extract(wid + 1)
    node_base = wid * NPT

    c0 = (e_start // CHUNK) * CHUNK
    nch = lax.max(jnp.int32(0), (e_end - c0 + (CHUNK - 1)) // CHUNK)

    nb_vec = jnp.full((LANES,), node_base, jnp.int32)
    nt_vec = jnp.full((LANES,), node_base + NPT, jnp.int32)

    buf_a = (r_a, ii_a, jj_a)
    buf_b = (r_b, ii_b, jj_b)

    def copies(bufs, sem, off):
        srcs = (r_hbm, ii_hbm, jj_hbm)
        return [pltpu.make_async_copy(s.at[pl.ds(off, CHUNK)], d, sem)
                for s, d in zip(srcs, bufs)]

    def issue(bufs, sem, off):
        for c in copies(bufs, sem, off):
            c.start()

    def drain(bufs, sem, off):
        for c in copies(bufs, sem, off):
            c.wait()

    def compute(bufs):
        r_v, ii_v, jj_v = bufs

        def one_vreg(v):
            ii = ii_v[pl.ds(v * LANES, LANES)]
            jj = jj_v[pl.ds(v * LANES, LANES)]
            r = r_v[pl.ds(v * LANES, LANES)]
            qi = plsc.load_gather(qa_v, [ii])
            qj = plsc.load_gather(qa_v, [jj])
            pw = _pair_term(r, qi, qj)
            m = (ii >= nb_vec) & (ii < nt_vec)
            loc = ii - nb_vec
            loc = jnp.clip(loc, jnp.int32(0), jnp.int32(ACC_PAD - 1))
            plsc.addupdate_scatter(acc_v, [loc], pw, mask=m)

        def vreg_body(v, carry2):
            for s in range(4):
                one_vreg(4 * v + s)
            return carry2
        lax.fori_loop(0, CHUNK // LANES // 4, vreg_body, 0)

    @pl.when(nch > 0)
    def _prime():
        issue(buf_a, sem_a, c0)

    def pair_body(mm, carry):
        k0 = 2 * mm
        k1 = k0 + 1
        off0 = c0 + k0 * CHUNK
        off1 = c0 + k1 * CHUNK

        @pl.when(k1 < nch)
        def _():
            issue(buf_b, sem_b, off1)

        drain(buf_a, sem_a, off0)
        compute(buf_a)

        @pl.when(k0 + 2 < nch)
        def _():
            issue(buf_a, sem_a, off0 + 2 * CHUNK)

        @pl.when(k1 < nch)
        def _():
            drain(buf_b, sem_b, off1)
            compute(buf_b)
        return carry
    lax.fori_loop(0, (nch + 1) // 2, pair_body, 0)

    pltpu.sync_copy(acc_v, out_hbm.at[wid])


@jax.jit
def kernel(Qa, rij, idx_i, idx_j):
    node_edges = jnp.arange(0, N_NODES + 1, NPT, dtype=jnp.int32)
    bounds = jnp.searchsorted(idx_i, node_edges).astype(jnp.int32)
    bounds = jnp.pad(bounds, (0, 48 - bounds.shape[0]))

    mesh = plsc.VectorSubcoreMesh(core_axis_name="c", subcore_axis_name="s")
    run = pl.kernel(
        _body,
        out_type=jax.ShapeDtypeStruct((NW, ACC_PAD), jnp.float32),
        mesh=mesh,
        compiler_params=pltpu.CompilerParams(needs_layout_passes=False),
        scratch_types=[
            pltpu.VMEM((N_NODES,), jnp.float32),
            pltpu.VMEM((ACC_PAD,), jnp.float32),
            pltpu.VMEM((48,), jnp.int32),
            pltpu.VMEM((CHUNK,), jnp.float32),
            pltpu.VMEM((CHUNK,), jnp.int32),
            pltpu.VMEM((CHUNK,), jnp.int32),
            pltpu.VMEM((CHUNK,), jnp.float32),
            pltpu.VMEM((CHUNK,), jnp.int32),
            pltpu.VMEM((CHUNK,), jnp.int32),
            pltpu.SemaphoreType.DMA,
            pltpu.SemaphoreType.DMA,
        ],
    )
    out2d = run(Qa, rij, idx_i, idx_j, bounds)
    return out2d[:, :NPT].reshape(-1)


# speedup vs baseline: 1.0074x; 1.0074x over previous
import jax
import jax.numpy as jnp
from jax import lax
from jax.experimental import pallas as pl
from jax.experimental.pallas import tpu as pltpu
from jax.experimental.pallas import tpu_sc as plsc

N_NODES = 100000
N_EDGES = 3200000
CUTOFF = 10.0
ON_CUT = 0.25 * CUTOFF
OFF_CUT = 0.75 * CUTOFF
ALPHA = 4.0 / CUTOFF + 0.001

NW = 32
NPT = N_NODES // NW
ACC_PAD = 3136
CHUNK = 3200
LANES = 16


def _pair_term(r, qi, qj):
    one = jnp.float32(1.0)
    x = (r - jnp.float32(ON_CUT)) * jnp.float32(1.0 / (OFF_CUT - ON_CUT))
    d = (x + x - one) / (x - x * x)
    d = jnp.minimum(d, jnp.float32(80.0))
    f = one / (one + jnp.exp(d))
    f = jnp.where(x <= 0.0, one, jnp.where(x >= 1.0, jnp.float32(0.0), f))
    rr = r * r
    u = rr + one
    ui = plsc.bitcast(u, jnp.int32)
    yi = jnp.int32(0x5F3759DF) - (ui >> 1)
    y = plsc.bitcast(yi, jnp.float32)
    y = y * (jnp.float32(1.5) - jnp.float32(0.5) * u * y * y)
    z = jnp.float32(ALPHA) * r
    ct = one / (r + jnp.float32(0.47047 * ALPHA) * rr)
    t = r * ct
    coul = (one + jnp.float32(0.47047) * z) * ct
    p = (jnp.float32(0.3480242)
         + t * (jnp.float32(-0.0958798) + t * jnp.float32(0.7478556)))
    erfc = t * p * jnp.exp(-z * z)
    return qi * qj * (coul + f * (y - coul)) * erfc


def _body(qa_hbm, r_hbm, ii_hbm, jj_hbm, bnd_hbm, out_hbm,
          qa_v, acc_v, bnd_v,
          r_a, ii_a, jj_a, r_b, ii_b, jj_b, sem_a, sem_b):
    cid = lax.axis_index("c")
    sid = lax.axis_index("s")
    wid = cid * 16 + sid

    pltpu.sync_copy(qa_hbm, qa_v)
    pltpu.sync_copy(bnd_hbm, bnd_v)

    def zero_body(i, carry):
        acc_v[pl.ds(i * LANES, LANES)] = jnp.zeros((LANES,), jnp.float32)
        return carry
    lax.fori_loop(0, ACC_PAD // LANES, zero_body, 0)

    lanes = lax.iota(jnp.int32, 16)

    def extract(pos):
        acc = jnp.zeros((LANES,), jnp.int32)
        for k in range(3):
            bk = bnd_v[pl.ds(k * LANES, LANES)]
            acc = acc + jnp.where(lanes + jnp.int32(k * LANES) == pos, bk,
                                  jnp.int32(0))
        return jnp.max(acc)

    e_start = extract(wid)
    e_end = extract(wid + 1)
    node_base = wid * NPT

    c0 = (e_start // CHUNK) * CHUNK
    nch = lax.max(jnp.int32(0), (e_end - c0 + (CHUNK - 1)) // CHUNK)

    nb_vec = jnp.full((LANES,), node_base, jnp.int32)
    nt_vec = jnp.full((LANES,), node_base + NPT, jnp.int32)

    buf_a = (r_a, ii_a, jj_a)
    buf_b = (r_b, ii_b, jj_b)

    def copies(bufs, sem, off):
        srcs = (r_hbm, ii_hbm, jj_hbm)
        return [pltpu.make_async_copy(s.at[pl.ds(off, CHUNK)], d, sem)
                for s, d in zip(srcs, bufs)]

    def issue(bufs, sem, off):
        for c in copies(bufs, sem, off):
            c.start()

    def drain(bufs, sem, off):
        for c in copies(bufs, sem, off):
            c.wait()

    def compute(bufs):
        r_v, ii_v, jj_v = bufs

        def one_vreg(v):
            ii = ii_v[pl.ds(v * LANES, LANES)]
            jj = jj_v[pl.ds(v * LANES, LANES)]
            r = r_v[pl.ds(v * LANES, LANES)]
            qi = plsc.load_gather(qa_v, [ii])
            qj = plsc.load_gather(qa_v, [jj])
            pw = _pair_term(r, qi, qj)
            m = (ii >= nb_vec) & (ii < nt_vec)
            loc = ii - nb_vec
            loc = jnp.clip(loc, jnp.int32(0), jnp.int32(ACC_PAD - 1))
            plsc.addupdate_scatter(acc_v, [loc], pw, mask=m)

        def vreg_body(v, carry2):
            for s in range(4):
                one_vreg(4 * v + s)
            return carry2
        lax.fori_loop(0, CHUNK // LANES // 4, vreg_body, 0)

    @pl.when(nch > 0)
    def _prime():
        issue(buf_a, sem_a, c0)

    def pair_body(mm, carry):
        k0 = 2 * mm
        k1 = k0 + 1
        off0 = c0 + k0 * CHUNK
        off1 = c0 + k1 * CHUNK

        @pl.when(k1 < nch)
        def _():
            issue(buf_b, sem_b, off1)

        drain(buf_a, sem_a, off0)
        compute(buf_a)

        @pl.when(k0 + 2 < nch)
        def _():
            issue(buf_a, sem_a, off0 + 2 * CHUNK)

        @pl.when(k1 < nch)
        def _():
            drain(buf_b, sem_b, off1)
            compute(buf_b)
        return carry
    lax.fori_loop(0, (nch + 1) // 2, pair_body, 0)

    pltpu.sync_copy(acc_v, out_hbm.at[wid])


@jax.jit
def kernel(Qa, rij, idx_i, idx_j):
    node_edges = jnp.arange(0, N_NODES + 1, NPT, dtype=jnp.int32)
    bounds = jnp.searchsorted(idx_i, node_edges).astype(jnp.int32)
    bounds = jnp.pad(bounds, (0, 48 - bounds.shape[0]))

    mesh = plsc.VectorSubcoreMesh(core_axis_name="c", subcore_axis_name="s")
    run = pl.kernel(
        _body,
        out_type=jax.ShapeDtypeStruct((NW, ACC_PAD), jnp.float32),
        mesh=mesh,
        compiler_params=pltpu.CompilerParams(needs_layout_passes=False),
        scratch_types=[
            pltpu.VMEM((N_NODES,), jnp.float32),
            pltpu.VMEM((ACC_PAD,), jnp.float32),
            pltpu.VMEM((48,), jnp.int32),
            pltpu.VMEM((CHUNK,), jnp.float32),
            pltpu.VMEM((CHUNK,), jnp.int32),
            pltpu.VMEM((CHUNK,), jnp.int32),
            pltpu.VMEM((CHUNK,), jnp.float32),
            pltpu.VMEM((CHUNK,), jnp.int32),
            pltpu.VMEM((CHUNK,), jnp.int32),
            pltpu.SemaphoreType.DMA,
            pltpu.SemaphoreType.DMA,
        ],
    )
    out2d = run(Qa, rij, idx_i, idx_j, bounds)
    return out2d[:, :NPT].reshape(-1)
